# Initial kernel scaffold; baseline (speedup 1.0000x reference)
import jax
import jax.numpy as jnp
from jax.experimental import pallas as pl


def kernel(router_logits):
    raise NotImplementedError("write your pallas kernel here")


# speedup vs baseline: 2.2972x; 2.2972x over previous
import functools

import jax
import jax.numpy as jnp
import numpy as np
from jax import lax
from jax.experimental import pallas as pl
from jax.experimental.pallas import tpu as pltpu
from jax.experimental.pallas import tpu_sc as plsc

T = 32768
E = 64
K = 8
EPS2 = np.float32(0.02)

NUM_CORES = 2
NUM_SUBCORES = 16
NW = NUM_CORES * NUM_SUBCORES
ROWS_PER_W = T // NW
CHUNK = 256
GROUPS = CHUNK // 16

_NEG_INF = np.float32(-np.inf)


def _router_body(logits_hbm, out_idx_hbm, out_val_hbm, l_v, oi_v, ov_v):
  wid = lax.axis_index("s") * NUM_CORES + lax.axis_index("c")
  base = wid * ROWS_PER_W
  lanes = jnp.arange(16, dtype=jnp.int32)

  for c in range(ROWS_PER_W // CHUNK):
    rbase = base + c * CHUNK
    pltpu.sync_copy(logits_hbm.at[pl.ds(rbase * E, CHUNK * E)], l_v)

    def group_body(g, _):
      rowsE = g * (16 * E) + lanes * E
      rowsK = g * (16 * K) + lanes * K

      def max_body(e, carry):
        mv, mi = carry
        v = plsc.load_gather(l_v, [rowsE + e])
        gt = v > mv
        ev = jnp.full((16,), e, dtype=jnp.int32)
        return jnp.where(gt, v, mv), jnp.where(gt, ev, mi)

      mv, mi = lax.fori_loop(
          0, E, max_body,
          (jnp.full((16,), _NEG_INF), jnp.zeros((16,), jnp.int32)))

      for k in range(K):
        def fused_body(e, carry, mv=mv, mi=mi):
          den, nmv, nmi = carry
          v = plsc.load_gather(l_v, [rowsE + e])
          drop = (mv - v) > EPS2 * jnp.maximum(jnp.abs(v), mv)
          term = jnp.where(drop, np.float32(0.0), jnp.exp(v - mv))
          ev = jnp.full((16,), e, dtype=jnp.int32)
          cand = (v > nmv) & (ev != mi)
          return (den + term, jnp.where(cand, v, nmv),
                  jnp.where(cand, ev, nmi))

        den, nmv, nmi = lax.fori_loop(
            0, E, fused_body,
            (jnp.zeros((16,), jnp.float32), jnp.full((16,), _NEG_INF),
             jnp.zeros((16,), jnp.int32)))

        plsc.store_scatter(oi_v, [rowsK + k], mi)
        plsc.store_scatter(ov_v, [rowsK + k], np.float32(1.0) / den)
        plsc.store_scatter(l_v, [rowsE + mi], jnp.full((16,), _NEG_INF))
        mv, mi = nmv, nmi
      return 0

    lax.fori_loop(0, GROUPS, group_body, 0)
    pltpu.sync_copy(oi_v, out_idx_hbm.at[pl.ds(rbase * K, CHUNK * K)])
    pltpu.sync_copy(ov_v, out_val_hbm.at[pl.ds(rbase * K, CHUNK * K)])


@jax.jit
def _router(router_logits):
  mesh = plsc.VectorSubcoreMesh(
      core_axis_name="c", subcore_axis_name="s", num_cores=NUM_CORES)
  f = functools.partial(
      pl.kernel,
      mesh=mesh,
      compiler_params=pltpu.CompilerParams(needs_layout_passes=False),
      out_type=[
          jax.ShapeDtypeStruct((T * K,), jnp.int32),
          jax.ShapeDtypeStruct((T * K,), jnp.float32),
      ],
      scratch_types=[
          pltpu.VMEM((CHUNK * E,), jnp.float32),
          pltpu.VMEM((CHUNK * K,), jnp.int32),
          pltpu.VMEM((CHUNK * K,), jnp.float32),
      ],
  )(_router_body)
  oi, ov = f(router_logits.reshape(T * E))
  return oi.reshape(T, K), ov.reshape(T, K)


def kernel(router_logits):
  return _router(router_logits)
